# baseline (speedup 1.0000x reference)
import functools

import jax
import jax.numpy as jnp
from jax import lax
from jax.experimental import pallas as pl
from jax.experimental.pallas import tpu as pltpu
from jax.experimental.pallas import tpu_sc as plsc

B = 16384
D = 32

_INFO = plsc.get_sparse_core_info()
NC = _INFO.num_cores
NS = _INFO.num_subcores
L = _INFO.num_lanes
NW = NC * NS
CHUNK = B // NW
JCH = 128
NJ = CHUNK // JCH

LN2 = 0.6931471805599453
SQRT2 = 1.4142135623730951


@functools.partial(
    pl.kernel,
    out_type=jax.ShapeDtypeStruct((B,), jnp.float32),
    mesh=plsc.VectorSubcoreMesh(core_axis_name="c", subcore_axis_name="s"),
    compiler_params=pltpu.CompilerParams(
        needs_layout_passes=False, use_tc_tiling_on_sc=False),
    scratch_types=[
        pltpu.VMEM((CHUNK,), jnp.int32),
        pltpu.VMEM((CHUNK,), jnp.int32),
        pltpu.VMEM((CHUNK,), jnp.float32),
        pltpu.VMEM((CHUNK, D), jnp.float32),
        pltpu.VMEM((CHUNK, D), jnp.float32),
        pltpu.VMEM((CHUNK,), jnp.float32),
        pltpu.VMEM((CHUNK,), jnp.float32),
        pltpu.VMEM((CHUNK,), jnp.float32),
        pltpu.VMEM((L,), jnp.float32),
        pltpu.VMEM((L,), jnp.float32),
        pltpu.SemaphoreType.DMA,
    ],
)
def _mfpoly2_sc(u_hbm, i_hbm, f_hbm, ub_hbm, uv_hbm, ib_hbm, iv_hbm,
                c0_hbm, c1_hbm, out_hbm,
                idx_u, idx_i, f_v, vu, vi, bu, bi, o_v, c0_v, c1_v, sem):
    wid = lax.axis_index("s") * NC + lax.axis_index("c")
    base = pl.multiple_of(wid * CHUNK, CHUNK)

    pltpu.sync_copy(u_hbm.at[pl.ds(base, CHUNK)], idx_u)
    pltpu.sync_copy(i_hbm.at[pl.ds(base, CHUNK)], idx_i)
    pltpu.sync_copy(f_hbm.at[pl.ds(base, CHUNK)], f_v)
    pltpu.sync_copy(c0_hbm, c0_v)
    pltpu.sync_copy(c1_hbm, c1_v)

    copies = []
    for j in range(NJ):
        sl = pl.ds(j * JCH, JCH)
        copies.append(pltpu.async_copy(uv_hbm.at[idx_u.at[sl]], vu.at[sl], sem))
        copies.append(pltpu.async_copy(iv_hbm.at[idx_i.at[sl]], vi.at[sl], sem))
        copies.append(pltpu.async_copy(ub_hbm.at[idx_u.at[sl]], bu.at[sl], sem))
        copies.append(pltpu.async_copy(ib_hbm.at[idx_i.at[sl]], bi.at[sl], sem))
    for c in copies:
        c.wait()

    c0s = c0_v[...]
    c1s = c1_v[...]

    def group(g, _):
        gbase = pl.multiple_of(g * L, L)
        rows = gbase + lax.iota(jnp.int32, L)
        acc = jnp.zeros((L,), jnp.float32)
        for d in range(D):
            cols = jnp.full((L,), d, jnp.int32)
            xu = plsc.load_gather(vu, [rows, cols])
            xi = plsc.load_gather(vi, [rows, cols])
            acc = acc + xu * xi

        sl = pl.ds(gbase, L)
        fg = f_v[sl]
        xb = plsc.bitcast(fg, jnp.int32)
        e = lax.shift_right_logical(xb, 23) - 127
        m = plsc.bitcast((xb & 0x7FFFFF) | (127 << 23), jnp.float32)
        big = m > SQRT2
        m = jnp.where(big, m * 0.5, m)
        e = jnp.where(big, e + 1, e)
        s = (m - 1.0) / (m + 1.0)
        z = s * s
        ln_m = s * (2.0 + z * (2.0 / 3.0 + z * (2.0 / 5.0
                    + z * (2.0 / 7.0 + z * (2.0 / 9.0)))))
        logf = ln_m + e.astype(jnp.float32) * LN2

        o_v[sl] = acc + bu[sl] + bi[sl] + c1s * logf + c0s
        return 0

    lax.fori_loop(0, CHUNK // L, group, 0)

    pltpu.sync_copy(o_v, out_hbm.at[pl.ds(base, CHUNK)])


def kernel(u, i, f, glob_bias, user_bias, user_vec, item_bias, item_vec,
           poly_W, poly_b):
    u = jnp.squeeze(u).astype(jnp.int32)
    i = jnp.squeeze(i).astype(jnp.int32)
    f = jnp.squeeze(f).astype(jnp.float32)
    c1 = jnp.full((L,), poly_W[0, 0] + 2.0 * poly_W[0, 1], jnp.float32)
    c0 = jnp.full((L,), poly_b[0] + glob_bias[0], jnp.float32)
    return _mfpoly2_sc(u, i, f, user_bias, user_vec, item_bias, item_vec,
                       c0, c1)


# speedup vs baseline: 1.4654x; 1.4654x over previous
import functools

import jax
import jax.numpy as jnp
from jax import lax
from jax.experimental import pallas as pl
from jax.experimental.pallas import tpu as pltpu
from jax.experimental.pallas import tpu_sc as plsc

B = 16384
D = 32

_INFO = plsc.get_sparse_core_info()
NC = _INFO.num_cores
NS = _INFO.num_subcores
L = _INFO.num_lanes
NW = NC * NS
CHUNK = B // NW
HALF = CHUNK // 2
JCH = 128
NJ = CHUNK // JCH
GA = 2

LN2 = 0.6931471805599453
SQRT2 = 1.4142135623730951


@functools.partial(
    pl.kernel,
    out_type=jax.ShapeDtypeStruct((B,), jnp.float32),
    mesh=plsc.VectorSubcoreMesh(core_axis_name="c", subcore_axis_name="s"),
    compiler_params=pltpu.CompilerParams(needs_layout_passes=False),
    scratch_types=[
        pltpu.VMEM((CHUNK,), jnp.int32),
        pltpu.VMEM((CHUNK,), jnp.int32),
        pltpu.VMEM((CHUNK,), jnp.float32),
        pltpu.VMEM((HALF, D), jnp.float32),
        pltpu.VMEM((HALF, D), jnp.float32),
        pltpu.VMEM((CHUNK,), jnp.float32),
        pltpu.VMEM((CHUNK,), jnp.float32),
        pltpu.VMEM((CHUNK,), jnp.float32),
        pltpu.VMEM((L,), jnp.float32),
        pltpu.VMEM((L,), jnp.float32),
        pltpu.SemaphoreType.DMA,
        pltpu.SemaphoreType.DMA,
        pltpu.SemaphoreType.DMA,
    ],
)
def _mfpoly2_sc(u_hbm, i_hbm, f_hbm, ub_hbm, uv_hbm, ib_hbm, iv_hbm,
                c0_hbm, c1_hbm, out_hbm,
                idx_u, idx_i, f_v, vu, vi, bu, bi, o_v, c0_v, c1_v,
                sem_b, sem_u, sem_i):
    wid = lax.axis_index("s") * NC + lax.axis_index("c")
    base = pl.multiple_of(wid * CHUNK, CHUNK)

    pltpu.sync_copy(u_hbm.at[pl.ds(base, CHUNK)], idx_u)
    pltpu.sync_copy(i_hbm.at[pl.ds(base, CHUNK)], idx_i)
    pltpu.sync_copy(f_hbm.at[pl.ds(base, CHUNK)], f_v)
    pltpu.sync_copy(c0_hbm, c0_v)
    pltpu.sync_copy(c1_hbm, c1_v)

    bias_copies = []
    for j in range(NJ):
        sl = pl.ds(j * JCH, JCH)
        bias_copies.append(
            pltpu.async_copy(ub_hbm.at[idx_u.at[sl]], bu.at[sl], sem_b))
        bias_copies.append(
            pltpu.async_copy(ib_hbm.at[idx_i.at[sl]], bi.at[sl], sem_b))

    c0s = c0_v[...]
    c1s = c1_v[...]

    def row_wait():
        pltpu.make_async_copy(uv_hbm.at[0], vu.at[0], sem_u).wait()
        pltpu.make_async_copy(iv_hbm.at[0], vi.at[0], sem_i).wait()

    for h in range(2):
        hoff = h * HALF

        def fire(g, _):
            gb = pl.multiple_of(g * L, L)
            uvec = idx_u[pl.ds(hoff + gb, L)]
            ivec = idx_i[pl.ds(hoff + gb, L)]
            for kk in range(L):
                pltpu.make_async_copy(
                    uv_hbm.at[uvec[kk]], vu.at[gb + kk], sem_u).start()
                pltpu.make_async_copy(
                    iv_hbm.at[ivec[kk]], vi.at[gb + kk], sem_i).start()

            @pl.when(g >= GA)
            def _():
                for kk in range(L):
                    row_wait()
            return 0

        lax.fori_loop(0, HALF // L, fire, 0)

        def drain(g, _):
            for kk in range(L):
                row_wait()
            return 0

        lax.fori_loop(0, GA, drain, 0)
        if h == 0:
            for c in bias_copies:
                c.wait()

        def group(g, _):
            gb = pl.multiple_of(g * L, L)
            rows = gb + lax.iota(jnp.int32, L)
            acc = jnp.zeros((L,), jnp.float32)
            for d in range(D):
                cols = jnp.full((L,), d, jnp.int32)
                xu = plsc.load_gather(vu, [rows, cols])
                xi = plsc.load_gather(vi, [rows, cols])
                acc = acc + xu * xi

            sl = pl.ds(pl.multiple_of(hoff + gb, L), L)
            fg = f_v[sl]
            xb = plsc.bitcast(fg, jnp.int32)
            e = lax.shift_right_logical(xb, 23) - 127
            m = plsc.bitcast((xb & 0x7FFFFF) | (127 << 23), jnp.float32)
            big = m > SQRT2
            m = jnp.where(big, m * 0.5, m)
            e = jnp.where(big, e + 1, e)
            s = (m - 1.0) / (m + 1.0)
            z = s * s
            ln_m = s * (2.0 + z * (2.0 / 3.0 + z * (2.0 / 5.0
                        + z * (2.0 / 7.0 + z * (2.0 / 9.0)))))
            logf = ln_m + e.astype(jnp.float32) * LN2

            o_v[sl] = acc + bu[sl] + bi[sl] + c1s * logf + c0s
            return 0

        lax.fori_loop(0, HALF // L, group, 0)

    pltpu.sync_copy(o_v, out_hbm.at[pl.ds(base, CHUNK)])


def kernel(u, i, f, glob_bias, user_bias, user_vec, item_bias, item_vec,
           poly_W, poly_b):
    u = jnp.squeeze(u).astype(jnp.int32)
    i = jnp.squeeze(i).astype(jnp.int32)
    f = jnp.squeeze(f).astype(jnp.float32)
    c1 = jnp.full((L,), poly_W[0, 0] + 2.0 * poly_W[0, 1], jnp.float32)
    c0 = jnp.full((L,), poly_b[0] + glob_bias[0], jnp.float32)
    return _mfpoly2_sc(u, i, f, user_bias, user_vec, item_bias, item_vec,
                       c0, c1)
